# Initial kernel scaffold; baseline (speedup 1.0000x reference)
import jax
import jax.numpy as jnp
from jax.experimental import pallas as pl


def kernel(x, W, b, codebook):
    raise NotImplementedError("write your pallas kernel here")


# speedup vs baseline: 2.8075x; 2.8075x over previous
import functools

import jax
import jax.numpy as jnp
from jax import lax
from jax.experimental import pallas as pl
from jax.experimental.pallas import tpu as pltpu
from jax.experimental.pallas import tpu_sc as plsc

_NB = 512
_CD = 64
_BSZ = 4
_TSZ = 512
_NTOK = _BSZ * _TSZ


def _stats_body(x_ref, w_ref, b_ref, kidx_ref, pen_ref, cper_ref, pper_ref,
                probs_acc, counts_acc):
    i = pl.program_id(0)
    xb = x_ref[0]
    w = w_ref[...]
    logits = jnp.dot(w, xb, preferred_element_type=jnp.float32)
    logits = logits + b_ref[0][:, None]

    m = jnp.max(logits, axis=0, keepdims=True)
    e = jnp.exp(logits - m)
    s = jnp.sum(e, axis=0, keepdims=True)
    psum = jnp.sum(e / s, axis=1)

    iota_v = lax.broadcasted_iota(jnp.int32, (_NB, _TSZ), 0)
    k = jnp.min(jnp.where(logits == m, iota_v, _NB), axis=0)
    kidx_ref[0, 0, :] = k
    cnt = jnp.sum((iota_v == k[None, :]).astype(jnp.float32), axis=1)

    @pl.when(i == 0)
    def _():
        probs_acc[0, :] = psum
        counts_acc[0, :] = cnt

    @pl.when(i > 0)
    def _():
        probs_acc[0, :] = probs_acc[0, :] + psum
        counts_acc[0, :] = counts_acc[0, :] + cnt

    @pl.when(i == _BSZ - 1)
    def _():
        n = jnp.float32(_NTOK)
        hp = counts_acc[0, :] / n
        ap = probs_acc[0, :] / n
        cper = jnp.exp(-jnp.sum(hp * jnp.log(hp + 1e-7)))
        pper = jnp.exp(-jnp.sum(ap * jnp.log(ap + 1e-7)))
        cper_ref[0, 0] = cper
        pper_ref[0, 0] = pper
        pen_ref[0, 0] = (jnp.float32(_NB) - pper) / jnp.float32(_NB)


def _stats_call(x, W, b2d, interpret=False):
    return pl.pallas_call(
        _stats_body,
        grid=(_BSZ,),
        in_specs=[
            pl.BlockSpec((1, _CD, _TSZ), lambda i: (i, 0, 0)),
            pl.BlockSpec((_NB, _CD), lambda i: (0, 0)),
            pl.BlockSpec((1, _NB), lambda i: (0, 0)),
        ],
        out_specs=[
            pl.BlockSpec((1, 1, _TSZ), lambda i: (i, 0, 0)),
            pl.BlockSpec((1, 1), lambda i: (0, 0), memory_space=pltpu.SMEM),
            pl.BlockSpec((1, 1), lambda i: (0, 0), memory_space=pltpu.SMEM),
            pl.BlockSpec((1, 1), lambda i: (0, 0), memory_space=pltpu.SMEM),
        ],
        out_shape=[
            jax.ShapeDtypeStruct((_BSZ, 1, _TSZ), jnp.int32),
            jax.ShapeDtypeStruct((1, 1), jnp.float32),
            jax.ShapeDtypeStruct((1, 1), jnp.float32),
            jax.ShapeDtypeStruct((1, 1), jnp.float32),
        ],
        scratch_shapes=[
            pltpu.VMEM((1, _NB), jnp.float32),
            pltpu.VMEM((1, _NB), jnp.float32),
        ],
        interpret=interpret,
    )(x, W, b2d)


_NCORES = 2
_NSUB = 16
_NW = _NCORES * _NSUB
_TPW = _NTOK // _NW


@functools.lru_cache(maxsize=None)
def _make_sc_gather():
    @functools.partial(
        pl.kernel,
        out_type=jax.ShapeDtypeStruct((_NTOK, _CD), jnp.float32),
        mesh=plsc.VectorSubcoreMesh(core_axis_name="c", subcore_axis_name="s"),
        compiler_params=pltpu.CompilerParams(use_tc_tiling_on_sc=False),
        scratch_types=[
            pltpu.VMEM((_TPW,), jnp.int32),
            pltpu.VMEM((_TPW, _CD), jnp.float32),
            pltpu.SemaphoreType.DMA,
        ],
    )
    def _sc_gather(table_hbm, idx_hbm, out_hbm, idx_v, rows_v, sem):
        wid = lax.axis_index("s") * _NCORES + lax.axis_index("c")
        base = wid * _TPW
        pltpu.sync_copy(idx_hbm.at[pl.ds(base, _TPW)], idx_v)
        pltpu.async_copy(table_hbm.at[idx_v], rows_v, sem).wait()
        pltpu.sync_copy(rows_v, out_hbm.at[pl.ds(base, _TPW)])

    return _sc_gather


def kernel(x, W, b, codebook):
    kidx, pen, cper, pper = _stats_call(x, W, b.reshape(1, _NB))
    rows = _make_sc_gather()(codebook.reshape(_NB, _CD), kidx.reshape(_NTOK))
    out = jnp.transpose(rows.reshape(_BSZ, _TSZ, _CD), (0, 2, 1))
    return out, pen[0, 0], cper[0, 0], pper[0, 0]
